# baseline (speedup 1.0000x reference)
import functools

import jax
import jax.numpy as jnp
from jax import lax
from jax.experimental import pallas as pl
from jax.experimental.pallas import tpu as pltpu
from jax.experimental.pallas import tpu_sc as plsc

_NUM_CORES = 2
_NUM_SUBCORES = 16
_NUM_WORKERS = _NUM_CORES * _NUM_SUBCORES
_RPC = 2
_SLOTS = 3
_SENT = 2**30


def kernel(token_ids, weights):
    batch, seq = token_ids.shape
    num_rows, dim = weights.shape

    half = lax.shift_right_logical(token_ids, 1)
    halfp = jnp.pad(half, ((0, 0), (0, 128 - seq)), constant_values=_SENT)
    ih3 = halfp.reshape(batch // _RPC, _RPC, 128)
    wpair = weights.reshape(num_rows // 2, 2 * dim)

    rows_per_w = batch // _NUM_WORKERS
    n_chunks = rows_per_w // _RPC
    win = 128

    mesh = plsc.VectorSubcoreMesh(core_axis_name="c", subcore_axis_name="s")

    scratch = (
        [pltpu.VMEM((_RPC * win, 2 * dim), jnp.float32) for _ in range(_SLOTS)]
        + [pltpu.VMEM((_RPC, 128), jnp.int32) for _ in range(_SLOTS)]
        + [pltpu.SemaphoreType.DMA for _ in range(3 * _SLOTS)]
    )

    @functools.partial(
        pl.kernel,
        mesh=mesh,
        out_type=jax.ShapeDtypeStruct((batch, seq, 2 * dim), weights.dtype),
        scratch_types=scratch,
    )
    def gather_kernel(table_hbm, ih_hbm, out_hbm, *scr):
        rvs = scr[0:_SLOTS]
        ivs = scr[_SLOTS : 2 * _SLOTS]
        isem = scr[2 * _SLOTS : 3 * _SLOTS]
        gsem = scr[3 * _SLOTS : 4 * _SLOTS]
        ssem = scr[4 * _SLOTS : 5 * _SLOTS]

        wid = lax.axis_index("s") * _NUM_CORES + lax.axis_index("c")
        chunk_base = wid * n_chunks

        def idx_copies(t, s):
            return (
                pltpu.make_async_copy(ih_hbm.at[chunk_base + t], ivs[s], isem[s]),
            )

        def gather_copies(t, s):
            return tuple(
                pltpu.make_async_copy(
                    table_hbm.at[
                        plsc.Indices(ivs[s].at[r], ignored_value=_SENT)
                    ],
                    rvs[s].at[pl.ds(r * win, win)],
                    gsem[s],
                )
                for r in range(_RPC)
            )

        def store_copies(t, s):
            row0 = (chunk_base + t) * _RPC
            return tuple(
                pltpu.make_async_copy(
                    rvs[s].at[pl.ds(r * win, seq)], out_hbm.at[row0 + r], ssem[s]
                )
                for r in range(_RPC)
            )

        def start(cs):
            for c in cs:
                c.start()

        def wait(cs):
            for c in cs:
                c.wait()

        start(idx_copies(0, 0))
        for t in range(_SLOTS):
            s = t
            wait(idx_copies(t, s))
            start(gather_copies(t, s))
            start(idx_copies(t + 1, (t + 1) % _SLOTS))
            if t >= 1:
                wait(gather_copies(t - 1, s - 1))
                start(store_copies(t - 1, s - 1))

        @pl.loop(1, n_chunks // _SLOTS)
        def _(k):
            t0 = k * _SLOTS
            for j in range(_SLOTS):
                t = t0 + j
                s = j
                pj = (j - 1) % _SLOTS
                wait(idx_copies(t, s))
                wait(store_copies(t - _SLOTS, s))
                start(gather_copies(t, s))
                start(idx_copies(t + 1, (j + 1) % _SLOTS))
                wait(gather_copies(t - 1, pj))
                start(store_copies(t - 1, pj))

        t = n_chunks - 1
        s = t % _SLOTS
        pj = (s - 1) % _SLOTS
        wait(idx_copies(t, s))
        wait(store_copies(t - _SLOTS, s))
        start(gather_copies(t, s))
        wait(gather_copies(t - 1, pj))
        start(store_copies(t - 1, pj))

        wait(gather_copies(t, s))
        start(store_copies(t, s))
        for u in range(n_chunks - _SLOTS + 1, n_chunks + 1):
            wait(store_copies(u - 1, (u - 1) % _SLOTS))

    pairs = gather_kernel(wpair, ih3)
    odd = lax.bitwise_and(token_ids, 1)[..., None] == 1
    return jnp.where(odd, pairs[..., dim:], pairs[..., :dim])
